# speedup vs baseline: 53.6386x; 53.6386x over previous
import math

import jax
import jax.numpy as jnp
from jax.experimental import pallas as pl
from jax.experimental.pallas import tpu as pltpu

_IMG = 800.0
_N = 20000
_NPAD = 20480
_K = 2000
_KPAD = 2048
_OUT = 1000
_THRESH = 0.7
_MIN_SIZE = 0.001
_LOG_MAX = math.log(1000.0 / 16)


def _decode_body(ax0, ay0, ax1, ay1, rdx, rdy, rdw, rdh, sc,
                 bx0, by0, bx1, by1, ms):
    width = ax1[...] - ax0[...]
    height = ay1[...] - ay0[...]
    cx = ax0[...] + width / 2
    cy = ay0[...] + height / 2
    dw = jnp.minimum(rdw[...], _LOG_MAX)
    dh = jnp.minimum(rdh[...], _LOG_MAX)
    px = cx + rdx[...] * width
    py = cy + rdy[...] * height
    pw = jnp.exp(dw) * width
    ph = jnp.exp(dh) * height
    x0 = jnp.clip(px - 0.5 * pw, 0.0, _IMG)
    y0 = jnp.clip(py - 0.5 * ph, 0.0, _IMG)
    x1 = jnp.clip(px + 0.5 * pw, 0.0, _IMG)
    y1 = jnp.clip(py + 0.5 * ph, 0.0, _IMG)
    valid = ((x1 - x0) >= _MIN_SIZE) & ((y1 - y0) >= _MIN_SIZE)
    bx0[...] = x0
    by0[...] = y0
    bx1[...] = x1
    by1[...] = y1
    ms[...] = jnp.where(valid, sc[...], -jnp.inf)


def _nms_body(x0c, y0c, x1c, y1c, x0r, y0r, x1r, y1r, keep_out, m_scr):
    area_r = (x1r[...] - x0r[...]) * (y1r[...] - y0r[...])
    cid = jax.lax.broadcasted_iota(jnp.int32, (256, _KPAD), 1)
    for t in range(_KPAD // 256):
        sl = pl.ds(t * 256, 256)
        tx0 = x0c[sl, :]
        ty0 = y0c[sl, :]
        tx1 = x1c[sl, :]
        ty1 = y1c[sl, :]
        area_c = (tx1 - tx0) * (ty1 - ty0)
        wx = jnp.clip(jnp.minimum(tx1, x1r[...]) - jnp.maximum(tx0, x0r[...]),
                      0.0, None)
        wy = jnp.clip(jnp.minimum(ty1, y1r[...]) - jnp.maximum(ty0, y0r[...]),
                      0.0, None)
        inter = wx * wy
        iou = inter / (area_c + area_r - inter + 1e-9)
        rid = t * 256 + jax.lax.broadcasted_iota(jnp.int32, (256, _KPAD), 0)
        m = (iou > _THRESH) & (rid < cid)
        m_scr[sl, :] = m.astype(jnp.bfloat16)

    def cond(carry):
        return carry[1]

    def body(carry):
        k, _ = carry
        kb = jnp.broadcast_to(k, (8, _KPAD)).astype(jnp.bfloat16)
        supp = jnp.dot(kb, m_scr[...], preferred_element_type=jnp.float32)
        k_new = jnp.where(supp[0:1, :] > 0.0, 0.0, 1.0)
        return k_new, jnp.any(k_new != k)

    k0 = jnp.ones((1, _KPAD), jnp.float32)
    k_fin, _ = jax.lax.while_loop(cond, body, (k0, True))
    keep_out[...] = k_fin


def _decode(cols):
    shp = jax.ShapeDtypeStruct((_NPAD // 128, 128), jnp.float32)
    return pl.pallas_call(
        _decode_body,
        out_shape=[shp] * 5,
    )(*cols)


def _nms(tb):
    colrefs = [tb[:, i:i + 1] for i in range(4)]
    rowrefs = [tb[:, i].reshape(1, _KPAD) for i in range(4)]
    return pl.pallas_call(
        _nms_body,
        out_shape=jax.ShapeDtypeStruct((1, _KPAD), jnp.float32),
        scratch_shapes=[pltpu.VMEM((_KPAD, _KPAD), jnp.bfloat16)],
    )(*colrefs, *rowrefs)


def kernel(anchors, regressions, scores):
    pad = _NPAD - _N

    def col(x):
        return jnp.pad(x, (0, pad)).reshape(_NPAD // 128, 128)

    cols = ([col(anchors[:, i]) for i in range(4)]
            + [col(regressions[:, i]) for i in range(4)]
            + [col(scores)])
    bx0, by0, bx1, by1, ms = _decode(cols)
    boxes = jnp.stack([bx0.reshape(-1), by0.reshape(-1),
                       bx1.reshape(-1), by1.reshape(-1)], axis=1)
    masked = ms.reshape(-1)

    top_scores, top_idx = jax.lax.top_k(masked, _K)
    top_boxes = boxes[top_idx]

    tb = jnp.concatenate([top_boxes, jnp.zeros((_KPAD - _K, 4), jnp.float32)])
    keep_f = _nms(tb).reshape(-1)[:_K]
    keep = keep_f > 0.5

    kept_scores = jnp.where(keep, top_scores, -jnp.inf)
    final_scores, sel = jax.lax.top_k(kept_scores, _OUT)
    final_boxes = top_boxes[sel]
    final_valid = jnp.isfinite(final_scores)
    out = jnp.concatenate([final_boxes, final_scores[:, None]], axis=1)
    return jnp.where(final_valid[:, None], out, 0.0)
